# baseline (speedup 1.0000x reference)
import jax
import jax.numpy as jnp
from jax import lax
from jax.experimental import pallas as pl
from jax.experimental.pallas import tpu as pltpu
from jax.experimental.pallas import tpu_sc as plsc

N = 10000
E = 320000
D = 128
NC, NS = 2, 16
NW = NC * NS
NPAD = 10240
EPW = 10240
EPAD = NW * EPW
K = 32
BPB = 2
NCH = EPAD // K
CH0 = 384
CH1 = NCH // NS - CH0
SUP = 8
EROWS = 80


def _ln_logit(x, a2, b2, w):
    ones = jnp.ones((1, D), jnp.float32)
    dn = (((1,), (1,)), ((), ()))
    m1 = lax.dot_general(x, ones, dn, preferred_element_type=jnp.float32)
    m2 = lax.dot_general(x * x, ones, dn, preferred_element_type=jnp.float32)
    wa = a2 * w
    t = lax.dot_general(x, wa, dn, preferred_element_type=jnp.float32)
    c1 = jnp.sum(wa, axis=1, keepdims=True)
    c0 = jnp.sum(b2 * w, axis=1, keepdims=True)
    mean = m1 * (1.0 / D)
    var = (m2 - mean * m1) * (1.0 / (D - 1))
    inv = 1.0 / (jnp.sqrt(var) + 1e-6)
    return jnp.tanh(inv * (t - mean * c1) + c0)


def _node_logits_body(h_ref, ha2_ref, hb2_ref, hw_ref, ta2_ref, tb2_ref,
                      tw_ref, eh_ref, et_ref):
    x = h_ref[...]
    eh_ref[...] = _ln_logit(x, ha2_ref[...], hb2_ref[...], hw_ref[...])
    et_ref[...] = _ln_logit(x, ta2_ref[...], tb2_ref[...], tw_ref[...])


def _edge_logits_body(r_ref, ra2_ref, rb2_ref, rw_ref, er_ref):
    er_ref[...] = _ln_logit(r_ref[...], ra2_ref[...], rb2_ref[...],
                            rw_ref[...])


def _sc_body(ei_hbm, er_hbm, eh_hbm, et_hbm, h_hbm,
             ftu_out, esum_out,
             eh_v, et_v, src_v, dst_v, er_v, w_v,
             r0, r1, esl_v, idx_v,
             ftu_sh, esum_sh, g0, g1, ssem):
    cid = lax.axis_index("c")
    sid = lax.axis_index("s")
    rows = (r0, r1)
    gsem = (g0, g1)
    base_ch = jnp.where(cid == 0, CH0 * sid, NS * CH0 + CH1 * sid)
    nch = jnp.where(cid == 0, CH0, CH1)
    nb = nch // BPB
    zero16 = jnp.zeros((16,), jnp.float32)
    z16i = jnp.zeros((16,), jnp.int32)
    iota16 = lax.iota(jnp.int32, 16)

    pltpu.sync_copy(eh_hbm, eh_v)
    pltpu.sync_copy(et_hbm, et_v)

    for c in range(BPB):
        def _zero_rows(i, _, c=c):
            for k in range(D // 16):
                rows[c][i, pl.ds(k * 16, 16)] = zero16
            return 0
        lax.fori_loop(0, K, _zero_rows, 0)

    def _zero_esl(i, _):
        for k in range(D // 16):
            esl_v[i, pl.ds(k * 16, 16)] = zero16
        return 0

    lax.fori_loop(0, EROWS, _zero_esl, 0)
    for k in range(EROWS // 16):
        idx_v[0, pl.ds(k * 16, 16)] = iota16 + (16 * k)

    for b in range(NPAD // NS // K):
        pltpu.sync_copy(r0, ftu_sh.at[pl.ds(sid * (NPAD // NS) + b * K,
                                            K), :])

    @pl.when(sid < EROWS // 8)
    def _():
        pltpu.sync_copy(r0.at[pl.ds(0, 8), :], esum_sh.at[pl.ds(sid * 8, 8), :])

    plsc.subcore_barrier()

    def _stage(t, q):
        sl = pl.ds(base_ch + t * SUP, SUP)
        a = pltpu.async_copy(ei_hbm.at[0, sl], src_v.at[q], ssem)
        b = pltpu.async_copy(ei_hbm.at[1, sl], dst_v.at[q], ssem)
        c = pltpu.async_copy(er_hbm.at[sl], er_v.at[q], ssem)
        del a, b, c

    def _wait_stage(q):
        pltpu.make_async_copy(ei_hbm.at[0, pl.ds(0, SUP)],
                              src_v.at[q], ssem).wait()
        pltpu.make_async_copy(ei_hbm.at[1, pl.ds(0, SUP)],
                              dst_v.at[q], ssem).wait()
        pltpu.make_async_copy(er_hbm.at[pl.ds(0, SUP)],
                              er_v.at[q], ssem).wait()

    _stage(0, 0)
    _wait_stage(0)
    for c in range(BPB):
        pltpu.async_copy(h_hbm.at[src_v.at[0, c]], rows[c], gsem[c])

    def _body(s, carry):
        t = s // 4
        q = t & 1
        m = s % 4
        qn = ((s + 1) // 4) & 1
        basen = BPB * ((s + 1) % 4)

        @pl.when((m == 0) & (s < nb - 4))
        def _():
            _stage(t + 1, 1 - q)

        @pl.when((m == 3) & (s < nb - 1))
        def _():
            _wait_stage(1 - q)

        for c in range(BPB):
            jj = BPB * m + c
            for g in range(K // 16):
                sl = pl.ds(g * 16, 16)
                sv = src_v[q, jj, sl]
                dv = dst_v[q, jj, sl]
                dvg = jnp.minimum(dv, N - 1)
                x = (plsc.load_gather(eh_v, [sv])
                     + plsc.load_gather(et_v, [dvg])
                     + er_v[q, jj, sl])
                x = jnp.where(x >= 0.0, x, 0.2 * x)
                w = jnp.exp(x)
                w_v[0, sl] = w
                plsc.addupdate_scatter(
                    esl_v,
                    [lax.shift_right_logical(dv, 7),
                     jnp.bitwise_and(dv, 127)],
                    w)
            pltpu.make_async_copy(h_hbm.at[src_v.at[q, jj]], rows[c],
                                  gsem[c]).wait()
            for g in range(K // 16):
                wvec = w_v[0, pl.ds(g * 16, 16)]
                for tt in range(16):
                    i = g * 16 + tt
                    wv = jnp.full((16,), wvec[tt], jnp.float32)
                    for k in range(D // 16):
                        sl = pl.ds(k * 16, 16)
                        rows[c][i, sl] = rows[c][i, sl] * wv
            pltpu.sync_copy(rows[c], ftu_sh.at[dst_v.at[q, jj]], add=True)

            @pl.when(s < nb - 1)
            def _(c=c, qn=qn, basen=basen):
                pltpu.async_copy(h_hbm.at[src_v.at[qn, basen + c]], rows[c],
                                 gsem[c])

        return carry

    lax.fori_loop(0, nb, _body, 0)

    pltpu.sync_copy(esl_v, esum_sh.at[idx_v.at[0]], add=True)
    plsc.subcore_barrier()

    for b in range(NPAD // NS // K):
        sl = pl.ds(sid * (NPAD // NS) + b * K, K)
        pltpu.sync_copy(ftu_sh.at[sl, :], r0)
        pltpu.sync_copy(r0, ftu_out.at[cid, sl, :])

    @pl.when(sid < EROWS // 8)
    def _():
        sl = pl.ds(sid * 8, 8)
        pltpu.sync_copy(esum_sh.at[sl, :], esl_v.at[pl.ds(0, 8), :])
        pltpu.sync_copy(esl_v.at[pl.ds(0, 8), :], esum_out.at[cid, sl, :])


def _make_sc_call():
    mesh = plsc.VectorSubcoreMesh(
        core_axis_name="c", subcore_axis_name="s",
        num_cores=NC, num_subcores=NS)

    return pl.kernel(
        _sc_body,
        out_type=[jax.ShapeDtypeStruct((NC, NPAD, D), jnp.float32),
                  jax.ShapeDtypeStruct((NC, EROWS, D), jnp.float32)],
        mesh=mesh,
        compiler_params=pltpu.CompilerParams(needs_layout_passes=False),
        scratch_types=[
            pltpu.VMEM((N,), jnp.float32),
            pltpu.VMEM((N,), jnp.float32),
            pltpu.VMEM((2, SUP, K), jnp.int32),
            pltpu.VMEM((2, SUP, K), jnp.int32),
            pltpu.VMEM((2, SUP, K), jnp.float32),
            pltpu.VMEM((1, K), jnp.float32),
            pltpu.VMEM((K, D), jnp.float32),
            pltpu.VMEM((K, D), jnp.float32),
            pltpu.VMEM((EROWS, D), jnp.float32),
            pltpu.VMEM((1, EROWS), jnp.int32),
            pltpu.VMEM_SHARED((NPAD, D), jnp.float32),
            pltpu.VMEM_SHARED((EROWS, D), jnp.float32),
            pltpu.SemaphoreType.DMA,
            pltpu.SemaphoreType.DMA,
            pltpu.SemaphoreType.DMA,
        ],
    )


def _final_body(ftu_ref, esum_ref, fcw_ref, fcb_ref, out_ref):
    B = 1024
    ftu = ftu_ref[0] + ftu_ref[1]
    es2 = esum_ref[0] + esum_ref[1]
    rr = lax.broadcasted_iota(jnp.int32, (B, B // D), 0) // D
    gsel = (rr == lax.broadcasted_iota(jnp.int32, (B, B // D), 1))
    grows = lax.dot_general(gsel.astype(jnp.float32), es2,
                            (((1,), (0,)), ((), ())),
                            preferred_element_type=jnp.float32)
    cc = lax.broadcasted_iota(jnp.int32, (B, D), 0) % D
    csel = (cc == lax.broadcasted_iota(jnp.int32, (B, D), 1))
    ones = jnp.ones((1, D), jnp.float32)
    es = lax.dot_general(jnp.where(csel, grows, 0.0), ones,
                         (((1,), (1,)), ((), ())),
                         preferred_element_type=jnp.float32)
    o = lax.dot_general(ftu, fcw_ref[...], (((1,), (1,)), ((), ())),
                        preferred_element_type=jnp.float32)
    o = o / jnp.maximum(es, 1e-20) + fcb_ref[...]
    norm = jnp.sqrt(jnp.sum(o * o, axis=1, keepdims=True))
    out_ref[...] = o / jnp.maximum(norm, 1e-12)


def kernel(h, r, edge_index, head_W, tail_W, rel_W, fc_W, fc_b,
           ha2, hb2, ta2, tb2, ra2, rb2):
    f32 = jnp.float32
    eh2, et2 = pl.pallas_call(
        _node_logits_body,
        grid=(N // 2000,),
        in_specs=[
            pl.BlockSpec((2000, D), lambda i: (i, 0)),
            pl.BlockSpec((1, D), lambda i: (0, 0)),
            pl.BlockSpec((1, D), lambda i: (0, 0)),
            pl.BlockSpec((1, D), lambda i: (0, 0)),
            pl.BlockSpec((1, D), lambda i: (0, 0)),
            pl.BlockSpec((1, D), lambda i: (0, 0)),
            pl.BlockSpec((1, D), lambda i: (0, 0)),
        ],
        out_specs=[pl.BlockSpec((2000, 1), lambda i: (i, 0))] * 2,
        out_shape=[jax.ShapeDtypeStruct((N, 1), f32)] * 2,
    )(h, ha2.reshape(1, D), hb2.reshape(1, D), head_W,
      ta2.reshape(1, D), tb2.reshape(1, D), tail_W)

    er2 = pl.pallas_call(
        _edge_logits_body,
        grid=(E // 4000,),
        in_specs=[
            pl.BlockSpec((4000, D), lambda i: (i, 0)),
            pl.BlockSpec((1, D), lambda i: (0, 0)),
            pl.BlockSpec((1, D), lambda i: (0, 0)),
            pl.BlockSpec((1, D), lambda i: (0, 0)),
        ],
        out_specs=pl.BlockSpec((4000, 1), lambda i: (i, 0)),
        out_shape=jax.ShapeDtypeStruct((E, 1), f32),
    )(r, ra2.reshape(1, D), rb2.reshape(1, D), rel_W)

    pad = EPAD - E
    ei_pad = jnp.concatenate(
        [jnp.zeros((1, pad), jnp.int32), jnp.full((1, pad), N, jnp.int32)])
    ei = jnp.concatenate([edge_index, ei_pad], axis=1).reshape(2, NCH, K)
    er = jnp.concatenate([er2, jnp.zeros((pad, 1), f32)]).reshape(NCH, K)

    ftu_p, esum_p = _make_sc_call()(ei, er, eh2.reshape(N), et2.reshape(N), h)

    out = pl.pallas_call(
        _final_body,
        grid=(NPAD // 1024,),
        in_specs=[
            pl.BlockSpec((NC, 1024, D), lambda i: (0, i, 0)),
            pl.BlockSpec((NC, 8, D), lambda i: (0, i, 0)),
            pl.BlockSpec((D, D), lambda i: (0, 0)),
            pl.BlockSpec((1, D), lambda i: (0, 0)),
        ],
        out_specs=pl.BlockSpec((1024, D), lambda i: (i, 0)),
        out_shape=jax.ShapeDtypeStruct((N, D), f32),
    )(ftu_p, esum_p, fc_W, fc_b.reshape(1, D))
    return out


# speedup vs baseline: 1.0174x; 1.0174x over previous
import jax
import jax.numpy as jnp
from jax import lax
from jax.experimental import pallas as pl
from jax.experimental.pallas import tpu as pltpu
from jax.experimental.pallas import tpu_sc as plsc

N = 10000
E = 320000
D = 128
NC, NS = 2, 16
NW = NC * NS
NPAD = 10240
EPW = 10240
EPAD = NW * EPW
K = 32
BPB = 2
NCH = EPAD // K
CH0 = 456
CH1 = NCH // NS - CH0
SUP = 8
EROWS = 80


def _ln_logit(x, a2, b2, w):
    ones = jnp.ones((1, D), jnp.float32)
    dn = (((1,), (1,)), ((), ()))
    m1 = lax.dot_general(x, ones, dn, preferred_element_type=jnp.float32)
    m2 = lax.dot_general(x * x, ones, dn, preferred_element_type=jnp.float32)
    wa = a2 * w
    t = lax.dot_general(x, wa, dn, preferred_element_type=jnp.float32)
    c1 = jnp.sum(wa, axis=1, keepdims=True)
    c0 = jnp.sum(b2 * w, axis=1, keepdims=True)
    mean = m1 * (1.0 / D)
    var = (m2 - mean * m1) * (1.0 / (D - 1))
    inv = 1.0 / (jnp.sqrt(var) + 1e-6)
    return jnp.tanh(inv * (t - mean * c1) + c0)


def _node_logits_body(h_ref, ha2_ref, hb2_ref, hw_ref, ta2_ref, tb2_ref,
                      tw_ref, eh_ref, et_ref):
    x = h_ref[...]
    eh_ref[...] = _ln_logit(x, ha2_ref[...], hb2_ref[...], hw_ref[...])
    et_ref[...] = _ln_logit(x, ta2_ref[...], tb2_ref[...], tw_ref[...])


def _edge_logits_body(r_ref, ra2_ref, rb2_ref, rw_ref, er_ref):
    er_ref[...] = _ln_logit(r_ref[...], ra2_ref[...], rb2_ref[...],
                            rw_ref[...])


def _sc_body(ei_hbm, er_hbm, eh_hbm, et_hbm, h_hbm,
             ftu_out, esum_out,
             eh_v, et_v, src_v, dst_v, er_v, w_v,
             r0, r1, esl_v, idx_v,
             ftu_sh, esum_sh, g0, g1, ssem):
    cid = lax.axis_index("c")
    sid = lax.axis_index("s")
    rows = (r0, r1)
    gsem = (g0, g1)
    base_ch = jnp.where(cid == 0, CH0 * sid, NS * CH0 + CH1 * sid)
    nch = jnp.where(cid == 0, CH0, CH1)
    nb = nch // BPB
    zero16 = jnp.zeros((16,), jnp.float32)
    z16i = jnp.zeros((16,), jnp.int32)
    iota16 = lax.iota(jnp.int32, 16)

    pltpu.sync_copy(eh_hbm, eh_v)
    pltpu.sync_copy(et_hbm, et_v)

    for c in range(BPB):
        def _zero_rows(i, _, c=c):
            for k in range(D // 16):
                rows[c][i, pl.ds(k * 16, 16)] = zero16
            return 0
        lax.fori_loop(0, K, _zero_rows, 0)

    def _zero_esl(i, _):
        for k in range(D // 16):
            esl_v[i, pl.ds(k * 16, 16)] = zero16
        return 0

    lax.fori_loop(0, EROWS, _zero_esl, 0)
    for k in range(EROWS // 16):
        idx_v[0, pl.ds(k * 16, 16)] = iota16 + (16 * k)

    for b in range(NPAD // NS // K):
        pltpu.sync_copy(r0, ftu_sh.at[pl.ds(sid * (NPAD // NS) + b * K,
                                            K), :])

    @pl.when(sid < EROWS // 8)
    def _():
        pltpu.sync_copy(r0.at[pl.ds(0, 8), :], esum_sh.at[pl.ds(sid * 8, 8), :])

    plsc.subcore_barrier()

    def _stage(t, q):
        sl = pl.ds(base_ch + t * SUP, SUP)
        a = pltpu.async_copy(ei_hbm.at[0, sl], src_v.at[q], ssem)
        b = pltpu.async_copy(ei_hbm.at[1, sl], dst_v.at[q], ssem)
        c = pltpu.async_copy(er_hbm.at[sl], er_v.at[q], ssem)
        del a, b, c

    def _wait_stage(q):
        pltpu.make_async_copy(ei_hbm.at[0, pl.ds(0, SUP)],
                              src_v.at[q], ssem).wait()
        pltpu.make_async_copy(ei_hbm.at[1, pl.ds(0, SUP)],
                              dst_v.at[q], ssem).wait()
        pltpu.make_async_copy(er_hbm.at[pl.ds(0, SUP)],
                              er_v.at[q], ssem).wait()

    _stage(0, 0)
    _wait_stage(0)
    for c in range(BPB):
        pltpu.async_copy(h_hbm.at[src_v.at[0, c]], rows[c], gsem[c])

    def _body(s, carry):
        t = s // 4
        q = t & 1
        m = s % 4
        qn = ((s + 1) // 4) & 1
        basen = BPB * ((s + 1) % 4)

        @pl.when((m == 0) & (s < nb - 4))
        def _():
            _stage(t + 1, 1 - q)

        @pl.when((m == 3) & (s < nb - 1))
        def _():
            _wait_stage(1 - q)

        for c in range(BPB):
            jj = BPB * m + c
            for g in range(K // 16):
                sl = pl.ds(g * 16, 16)
                sv = src_v[q, jj, sl]
                dv = dst_v[q, jj, sl]
                dvg = jnp.minimum(dv, N - 1)
                x = (plsc.load_gather(eh_v, [sv])
                     + plsc.load_gather(et_v, [dvg])
                     + er_v[q, jj, sl])
                x = jnp.where(x >= 0.0, x, 0.2 * x)
                w = jnp.exp(x)
                w_v[0, sl] = w
                plsc.addupdate_scatter(
                    esl_v,
                    [lax.shift_right_logical(dv, 7),
                     jnp.bitwise_and(dv, 127)],
                    w)
            pltpu.make_async_copy(h_hbm.at[src_v.at[q, jj]], rows[c],
                                  gsem[c]).wait()
            for g in range(K // 16):
                wvec = w_v[0, pl.ds(g * 16, 16)]
                for tt in range(16):
                    i = g * 16 + tt
                    wv = jnp.full((16,), wvec[tt], jnp.float32)
                    for k in range(D // 16):
                        sl = pl.ds(k * 16, 16)
                        rows[c][i, sl] = rows[c][i, sl] * wv
            pltpu.sync_copy(rows[c], ftu_sh.at[dst_v.at[q, jj]], add=True)

            @pl.when(s < nb - 1)
            def _(c=c, qn=qn, basen=basen):
                pltpu.async_copy(h_hbm.at[src_v.at[qn, basen + c]], rows[c],
                                 gsem[c])

        return carry

    lax.fori_loop(0, nb, _body, 0)

    pltpu.sync_copy(esl_v, esum_sh.at[idx_v.at[0]], add=True)
    plsc.subcore_barrier()

    for b in range(NPAD // NS // K):
        sl = pl.ds(sid * (NPAD // NS) + b * K, K)
        pltpu.sync_copy(ftu_sh.at[sl, :], r0)
        pltpu.sync_copy(r0, ftu_out.at[cid, sl, :])

    @pl.when(sid < EROWS // 8)
    def _():
        sl = pl.ds(sid * 8, 8)
        pltpu.sync_copy(esum_sh.at[sl, :], esl_v.at[pl.ds(0, 8), :])
        pltpu.sync_copy(esl_v.at[pl.ds(0, 8), :], esum_out.at[cid, sl, :])


def _make_sc_call():
    mesh = plsc.VectorSubcoreMesh(
        core_axis_name="c", subcore_axis_name="s",
        num_cores=NC, num_subcores=NS)

    return pl.kernel(
        _sc_body,
        out_type=[jax.ShapeDtypeStruct((NC, NPAD, D), jnp.float32),
                  jax.ShapeDtypeStruct((NC, EROWS, D), jnp.float32)],
        mesh=mesh,
        compiler_params=pltpu.CompilerParams(needs_layout_passes=False),
        scratch_types=[
            pltpu.VMEM((N,), jnp.float32),
            pltpu.VMEM((N,), jnp.float32),
            pltpu.VMEM((2, SUP, K), jnp.int32),
            pltpu.VMEM((2, SUP, K), jnp.int32),
            pltpu.VMEM((2, SUP, K), jnp.float32),
            pltpu.VMEM((1, K), jnp.float32),
            pltpu.VMEM((K, D), jnp.float32),
            pltpu.VMEM((K, D), jnp.float32),
            pltpu.VMEM((EROWS, D), jnp.float32),
            pltpu.VMEM((1, EROWS), jnp.int32),
            pltpu.VMEM_SHARED((NPAD, D), jnp.float32),
            pltpu.VMEM_SHARED((EROWS, D), jnp.float32),
            pltpu.SemaphoreType.DMA,
            pltpu.SemaphoreType.DMA,
            pltpu.SemaphoreType.DMA,
        ],
    )


def _final_body(ftu_ref, esum_ref, fcw_ref, fcb_ref, out_ref):
    B = 1024
    ftu = ftu_ref[0] + ftu_ref[1]
    es2 = esum_ref[0] + esum_ref[1]
    rr = lax.broadcasted_iota(jnp.int32, (B, B // D), 0) // D
    gsel = (rr == lax.broadcasted_iota(jnp.int32, (B, B // D), 1))
    grows = lax.dot_general(gsel.astype(jnp.float32), es2,
                            (((1,), (0,)), ((), ())),
                            preferred_element_type=jnp.float32)
    cc = lax.broadcasted_iota(jnp.int32, (B, D), 0) % D
    csel = (cc == lax.broadcasted_iota(jnp.int32, (B, D), 1))
    ones = jnp.ones((1, D), jnp.float32)
    es = lax.dot_general(jnp.where(csel, grows, 0.0), ones,
                         (((1,), (1,)), ((), ())),
                         preferred_element_type=jnp.float32)
    o = lax.dot_general(ftu, fcw_ref[...], (((1,), (1,)), ((), ())),
                        preferred_element_type=jnp.float32)
    o = o / jnp.maximum(es, 1e-20) + fcb_ref[...]
    norm = jnp.sqrt(jnp.sum(o * o, axis=1, keepdims=True))
    out_ref[...] = o / jnp.maximum(norm, 1e-12)


def kernel(h, r, edge_index, head_W, tail_W, rel_W, fc_W, fc_b,
           ha2, hb2, ta2, tb2, ra2, rb2):
    f32 = jnp.float32
    eh2, et2 = pl.pallas_call(
        _node_logits_body,
        grid=(N // 2000,),
        in_specs=[
            pl.BlockSpec((2000, D), lambda i: (i, 0)),
            pl.BlockSpec((1, D), lambda i: (0, 0)),
            pl.BlockSpec((1, D), lambda i: (0, 0)),
            pl.BlockSpec((1, D), lambda i: (0, 0)),
            pl.BlockSpec((1, D), lambda i: (0, 0)),
            pl.BlockSpec((1, D), lambda i: (0, 0)),
            pl.BlockSpec((1, D), lambda i: (0, 0)),
        ],
        out_specs=[pl.BlockSpec((2000, 1), lambda i: (i, 0))] * 2,
        out_shape=[jax.ShapeDtypeStruct((N, 1), f32)] * 2,
    )(h, ha2.reshape(1, D), hb2.reshape(1, D), head_W,
      ta2.reshape(1, D), tb2.reshape(1, D), tail_W)

    er2 = pl.pallas_call(
        _edge_logits_body,
        grid=(E // 4000,),
        in_specs=[
            pl.BlockSpec((4000, D), lambda i: (i, 0)),
            pl.BlockSpec((1, D), lambda i: (0, 0)),
            pl.BlockSpec((1, D), lambda i: (0, 0)),
            pl.BlockSpec((1, D), lambda i: (0, 0)),
        ],
        out_specs=pl.BlockSpec((4000, 1), lambda i: (i, 0)),
        out_shape=jax.ShapeDtypeStruct((E, 1), f32),
    )(r, ra2.reshape(1, D), rb2.reshape(1, D), rel_W)

    pad = EPAD - E
    ei_pad = jnp.concatenate(
        [jnp.zeros((1, pad), jnp.int32), jnp.full((1, pad), N, jnp.int32)])
    ei = jnp.concatenate([edge_index, ei_pad], axis=1).reshape(2, NCH, K)
    er = jnp.concatenate([er2, jnp.zeros((pad, 1), f32)]).reshape(NCH, K)

    ftu_p, esum_p = _make_sc_call()(ei, er, eh2.reshape(N), et2.reshape(N), h)

    out = pl.pallas_call(
        _final_body,
        grid=(NPAD // 1024,),
        in_specs=[
            pl.BlockSpec((NC, 1024, D), lambda i: (0, i, 0)),
            pl.BlockSpec((NC, 8, D), lambda i: (0, i, 0)),
            pl.BlockSpec((D, D), lambda i: (0, 0)),
            pl.BlockSpec((1, D), lambda i: (0, 0)),
        ],
        out_specs=pl.BlockSpec((1024, D), lambda i: (i, 0)),
        out_shape=jax.ShapeDtypeStruct((N, D), f32),
    )(ftu_p, esum_p, fc_W, fc_b.reshape(1, D))
    return out


# speedup vs baseline: 1.0735x; 1.0551x over previous
import jax
import jax.numpy as jnp
from jax import lax
from jax.experimental import pallas as pl
from jax.experimental.pallas import tpu as pltpu
from jax.experimental.pallas import tpu_sc as plsc

N = 10000
E = 320000
D = 128
NC, NS = 2, 16
NW = NC * NS
NPAD = 10240
EPW = 10240
EPAD = NW * EPW
K = 32
BPB = 2
NCH = EPAD // K
CH0 = 456
CH1 = NCH // NS - CH0
SUP = 8
EROWS = 80


def _ln_logit(x, a2, b2, w):
    ones = jnp.ones((1, D), jnp.float32)
    dn = (((1,), (1,)), ((), ()))
    wa = a2 * w
    rhs = jnp.concatenate([ones, wa, jnp.zeros((6, D), jnp.float32)], axis=0)
    x2 = x * x
    t8 = lax.dot_general(x, rhs, dn, preferred_element_type=jnp.float32)
    m2 = lax.dot_general(x2, rhs, dn,
                         preferred_element_type=jnp.float32)[:, 0:1]
    m1 = t8[:, 0:1]
    t = t8[:, 1:2]
    c1 = jnp.sum(wa, axis=1, keepdims=True)
    c0 = jnp.sum(b2 * w, axis=1, keepdims=True)
    mean = m1 * (1.0 / D)
    var = (m2 - mean * m1) * (1.0 / (D - 1))
    inv = 1.0 / (jnp.sqrt(var) + 1e-6)
    return jnp.tanh(inv * (t - mean * c1) + c0)


def _node_logits_body(h_ref, ha2_ref, hb2_ref, hw_ref, ta2_ref, tb2_ref,
                      tw_ref, eh_ref, et_ref):
    x = h_ref[...]
    eh_ref[...] = _ln_logit(x, ha2_ref[...], hb2_ref[...], hw_ref[...])
    et_ref[...] = _ln_logit(x, ta2_ref[...], tb2_ref[...], tw_ref[...])


def _edge_logits_body(r_ref, ra2_ref, rb2_ref, rw_ref, er_ref):
    er_ref[...] = _ln_logit(r_ref[...], ra2_ref[...], rb2_ref[...],
                            rw_ref[...])


def _sc_body(ei_hbm, er_hbm, eh_hbm, et_hbm, h_hbm,
             ftu_out, esum_out,
             eh_v, et_v, src_v, dst_v, er_v, w_v,
             r0, r1, esl_v, idx_v,
             ftu_sh, esum_sh, g0, g1, ssem):
    cid = lax.axis_index("c")
    sid = lax.axis_index("s")
    rows = (r0, r1)
    gsem = (g0, g1)
    base_ch = jnp.where(cid == 0, CH0 * sid, NS * CH0 + CH1 * sid)
    nch = jnp.where(cid == 0, CH0, CH1)
    nb = nch // BPB
    zero16 = jnp.zeros((16,), jnp.float32)
    z16i = jnp.zeros((16,), jnp.int32)
    iota16 = lax.iota(jnp.int32, 16)

    pltpu.sync_copy(eh_hbm, eh_v)
    pltpu.sync_copy(et_hbm, et_v)

    for c in range(BPB):
        def _zero_rows(i, _, c=c):
            for k in range(D // 16):
                rows[c][i, pl.ds(k * 16, 16)] = zero16
            return 0
        lax.fori_loop(0, K, _zero_rows, 0)

    def _zero_esl(i, _):
        for k in range(D // 16):
            esl_v[i, pl.ds(k * 16, 16)] = zero16
        return 0

    lax.fori_loop(0, EROWS, _zero_esl, 0)
    for k in range(EROWS // 16):
        idx_v[0, pl.ds(k * 16, 16)] = iota16 + (16 * k)

    for b in range(NPAD // NS // K):
        pltpu.sync_copy(r0, ftu_sh.at[pl.ds(sid * (NPAD // NS) + b * K,
                                            K), :])

    @pl.when(sid < EROWS // 8)
    def _():
        pltpu.sync_copy(r0.at[pl.ds(0, 8), :], esum_sh.at[pl.ds(sid * 8, 8), :])

    plsc.subcore_barrier()

    def _stage(t, q):
        sl = pl.ds(base_ch + t * SUP, SUP)
        a = pltpu.async_copy(ei_hbm.at[0, sl], src_v.at[q], ssem)
        b = pltpu.async_copy(ei_hbm.at[1, sl], dst_v.at[q], ssem)
        c = pltpu.async_copy(er_hbm.at[sl], er_v.at[q], ssem)
        del a, b, c

    def _wait_stage(q):
        pltpu.make_async_copy(ei_hbm.at[0, pl.ds(0, SUP)],
                              src_v.at[q], ssem).wait()
        pltpu.make_async_copy(ei_hbm.at[1, pl.ds(0, SUP)],
                              dst_v.at[q], ssem).wait()
        pltpu.make_async_copy(er_hbm.at[pl.ds(0, SUP)],
                              er_v.at[q], ssem).wait()

    _stage(0, 0)
    _wait_stage(0)
    for c in range(BPB):
        pltpu.async_copy(h_hbm.at[src_v.at[0, c]], rows[c], gsem[c])

    def _body(s, carry):
        t = s // 4
        q = t & 1
        m = s % 4
        qn = ((s + 1) // 4) & 1
        basen = BPB * ((s + 1) % 4)

        @pl.when((m == 0) & (s < nb - 4))
        def _():
            _stage(t + 1, 1 - q)

        @pl.when((m == 3) & (s < nb - 1))
        def _():
            _wait_stage(1 - q)

        for c in range(BPB):
            jj = BPB * m + c
            for g in range(K // 16):
                sl = pl.ds(g * 16, 16)
                sv = src_v[q, jj, sl]
                dv = dst_v[q, jj, sl]
                dvg = jnp.minimum(dv, N - 1)
                x = (plsc.load_gather(eh_v, [sv])
                     + plsc.load_gather(et_v, [dvg])
                     + er_v[q, jj, sl])
                x = jnp.where(x >= 0.0, x, 0.2 * x)
                w = jnp.exp(x)
                w_v[0, sl] = w
                plsc.addupdate_scatter(
                    esl_v,
                    [lax.shift_right_logical(dv, 7),
                     jnp.bitwise_and(dv, 127)],
                    w)
            pltpu.make_async_copy(h_hbm.at[src_v.at[q, jj]], rows[c],
                                  gsem[c]).wait()
            for g in range(K // 16):
                wvec = w_v[0, pl.ds(g * 16, 16)]
                for tt in range(16):
                    i = g * 16 + tt
                    wv = jnp.full((16,), wvec[tt], jnp.float32)
                    for k in range(D // 16):
                        sl = pl.ds(k * 16, 16)
                        rows[c][i, sl] = rows[c][i, sl] * wv
            pltpu.sync_copy(rows[c], ftu_sh.at[dst_v.at[q, jj]], add=True)

            @pl.when(s < nb - 1)
            def _(c=c, qn=qn, basen=basen):
                pltpu.async_copy(h_hbm.at[src_v.at[qn, basen + c]], rows[c],
                                 gsem[c])

        return carry

    lax.fori_loop(0, nb, _body, 0)

    pltpu.sync_copy(esl_v, esum_sh.at[idx_v.at[0]], add=True)
    plsc.subcore_barrier()

    for b in range(NPAD // NS // K):
        sl = pl.ds(sid * (NPAD // NS) + b * K, K)
        pltpu.sync_copy(ftu_sh.at[sl, :], r0)
        pltpu.sync_copy(r0, ftu_out.at[cid, sl, :])

    @pl.when(sid < EROWS // 8)
    def _():
        sl = pl.ds(sid * 8, 8)
        pltpu.sync_copy(esum_sh.at[sl, :], esl_v.at[pl.ds(0, 8), :])
        pltpu.sync_copy(esl_v.at[pl.ds(0, 8), :], esum_out.at[cid, sl, :])


def _make_sc_call():
    mesh = plsc.VectorSubcoreMesh(
        core_axis_name="c", subcore_axis_name="s",
        num_cores=NC, num_subcores=NS)

    return pl.kernel(
        _sc_body,
        out_type=[jax.ShapeDtypeStruct((NC, NPAD, D), jnp.float32),
                  jax.ShapeDtypeStruct((NC, EROWS, D), jnp.float32)],
        mesh=mesh,
        compiler_params=pltpu.CompilerParams(needs_layout_passes=False),
        scratch_types=[
            pltpu.VMEM((N,), jnp.float32),
            pltpu.VMEM((N,), jnp.float32),
            pltpu.VMEM((2, SUP, K), jnp.int32),
            pltpu.VMEM((2, SUP, K), jnp.int32),
            pltpu.VMEM((2, SUP, K), jnp.float32),
            pltpu.VMEM((1, K), jnp.float32),
            pltpu.VMEM((K, D), jnp.float32),
            pltpu.VMEM((K, D), jnp.float32),
            pltpu.VMEM((EROWS, D), jnp.float32),
            pltpu.VMEM((1, EROWS), jnp.int32),
            pltpu.VMEM_SHARED((NPAD, D), jnp.float32),
            pltpu.VMEM_SHARED((EROWS, D), jnp.float32),
            pltpu.SemaphoreType.DMA,
            pltpu.SemaphoreType.DMA,
            pltpu.SemaphoreType.DMA,
        ],
    )


def _final_body(ftu_ref, esum_ref, fcw_ref, fcb_ref, out_ref):
    B = 1024
    ftu = ftu_ref[0] + ftu_ref[1]
    es2 = esum_ref[0] + esum_ref[1]
    rr = lax.broadcasted_iota(jnp.int32, (B, B // D), 0) // D
    gsel = (rr == lax.broadcasted_iota(jnp.int32, (B, B // D), 1))
    grows = lax.dot_general(gsel.astype(jnp.float32), es2,
                            (((1,), (0,)), ((), ())),
                            preferred_element_type=jnp.float32)
    cc = lax.broadcasted_iota(jnp.int32, (B, D), 0) % D
    csel = (cc == lax.broadcasted_iota(jnp.int32, (B, D), 1))
    ones = jnp.ones((1, D), jnp.float32)
    es = lax.dot_general(jnp.where(csel, grows, 0.0), ones,
                         (((1,), (1,)), ((), ())),
                         preferred_element_type=jnp.float32)
    o = lax.dot_general(ftu, fcw_ref[...], (((1,), (1,)), ((), ())),
                        preferred_element_type=jnp.float32)
    o = o / jnp.maximum(es, 1e-20) + fcb_ref[...]
    norm = jnp.sqrt(jnp.sum(o * o, axis=1, keepdims=True))
    out_ref[...] = o / jnp.maximum(norm, 1e-12)


def kernel(h, r, edge_index, head_W, tail_W, rel_W, fc_W, fc_b,
           ha2, hb2, ta2, tb2, ra2, rb2):
    f32 = jnp.float32
    eh2, et2 = pl.pallas_call(
        _node_logits_body,
        grid=(N // 2000,),
        in_specs=[
            pl.BlockSpec((2000, D), lambda i: (i, 0)),
            pl.BlockSpec((1, D), lambda i: (0, 0)),
            pl.BlockSpec((1, D), lambda i: (0, 0)),
            pl.BlockSpec((1, D), lambda i: (0, 0)),
            pl.BlockSpec((1, D), lambda i: (0, 0)),
            pl.BlockSpec((1, D), lambda i: (0, 0)),
            pl.BlockSpec((1, D), lambda i: (0, 0)),
        ],
        out_specs=[pl.BlockSpec((2000, 1), lambda i: (i, 0))] * 2,
        out_shape=[jax.ShapeDtypeStruct((N, 1), f32)] * 2,
    )(h, ha2.reshape(1, D), hb2.reshape(1, D), head_W,
      ta2.reshape(1, D), tb2.reshape(1, D), tail_W)

    er2 = pl.pallas_call(
        _edge_logits_body,
        grid=(E // 4000,),
        in_specs=[
            pl.BlockSpec((4000, D), lambda i: (i, 0)),
            pl.BlockSpec((1, D), lambda i: (0, 0)),
            pl.BlockSpec((1, D), lambda i: (0, 0)),
            pl.BlockSpec((1, D), lambda i: (0, 0)),
        ],
        out_specs=pl.BlockSpec((4000, 1), lambda i: (i, 0)),
        out_shape=jax.ShapeDtypeStruct((E, 1), f32),
    )(r, ra2.reshape(1, D), rb2.reshape(1, D), rel_W)

    pad = EPAD - E
    ei_pad = jnp.concatenate(
        [jnp.zeros((1, pad), jnp.int32), jnp.full((1, pad), N, jnp.int32)])
    ei = jnp.concatenate([edge_index, ei_pad], axis=1).reshape(2, NCH, K)
    er = jnp.concatenate([er2, jnp.zeros((pad, 1), f32)]).reshape(NCH, K)

    ftu_p, esum_p = _make_sc_call()(ei, er, eh2.reshape(N), et2.reshape(N), h)

    out = pl.pallas_call(
        _final_body,
        grid=(NPAD // 1024,),
        in_specs=[
            pl.BlockSpec((NC, 1024, D), lambda i: (0, i, 0)),
            pl.BlockSpec((NC, 8, D), lambda i: (0, i, 0)),
            pl.BlockSpec((D, D), lambda i: (0, 0)),
            pl.BlockSpec((1, D), lambda i: (0, 0)),
        ],
        out_specs=pl.BlockSpec((1024, D), lambda i: (i, 0)),
        out_shape=jax.ShapeDtypeStruct((N, D), f32),
    )(ftu_p, esum_p, fc_W, fc_b.reshape(1, D))
    return out


# speedup vs baseline: 1.0772x; 1.0034x over previous
import jax
import jax.numpy as jnp
from jax import lax
from jax.experimental import pallas as pl
from jax.experimental.pallas import tpu as pltpu
from jax.experimental.pallas import tpu_sc as plsc

N = 10000
E = 320000
D = 128
NC, NS = 2, 16
NW = NC * NS
NPAD = 10240
EPW = 10240
EPAD = NW * EPW
K = 32
BPB = 2
NCH = EPAD // K
CH0 = 480
CH1 = NCH // NS - CH0
SUP = 8
EROWS = 80


def _ln_logit(x, a2, b2, w):
    ones = jnp.ones((1, D), jnp.float32)
    dn = (((1,), (1,)), ((), ()))
    wa = a2 * w
    rhs = jnp.concatenate([ones, wa, jnp.zeros((6, D), jnp.float32)], axis=0)
    x2 = x * x
    t8 = lax.dot_general(x, rhs, dn, preferred_element_type=jnp.float32)
    m2 = lax.dot_general(x2, rhs, dn,
                         preferred_element_type=jnp.float32)[:, 0:1]
    m1 = t8[:, 0:1]
    t = t8[:, 1:2]
    c1 = jnp.sum(wa, axis=1, keepdims=True)
    c0 = jnp.sum(b2 * w, axis=1, keepdims=True)
    mean = m1 * (1.0 / D)
    var = (m2 - mean * m1) * (1.0 / (D - 1))
    inv = 1.0 / (jnp.sqrt(var) + 1e-6)
    return jnp.tanh(inv * (t - mean * c1) + c0)


def _node_logits_body(h_ref, ha2_ref, hb2_ref, hw_ref, ta2_ref, tb2_ref,
                      tw_ref, eh_ref, et_ref):
    x = h_ref[...]
    eh_ref[...] = _ln_logit(x, ha2_ref[...], hb2_ref[...], hw_ref[...])
    et_ref[...] = _ln_logit(x, ta2_ref[...], tb2_ref[...], tw_ref[...])


def _edge_logits_body(r_ref, ra2_ref, rb2_ref, rw_ref, er_ref):
    er_ref[...] = _ln_logit(r_ref[...], ra2_ref[...], rb2_ref[...],
                            rw_ref[...])


def _sc_body(ei_hbm, er_hbm, eh_hbm, et_hbm, h_hbm,
             ftu_out, esum_out,
             eh_v, et_v, src_v, dst_v, er_v, w_v,
             r0, r1, esl_v, idx_v,
             ftu_sh, esum_sh, g0, g1, ssem):
    cid = lax.axis_index("c")
    sid = lax.axis_index("s")
    rows = (r0, r1)
    gsem = (g0, g1)
    base_ch = jnp.where(cid == 0, CH0 * sid, NS * CH0 + CH1 * sid)
    nch = jnp.where(cid == 0, CH0, CH1)
    nb = nch // BPB
    zero16 = jnp.zeros((16,), jnp.float32)
    z16i = jnp.zeros((16,), jnp.int32)
    iota16 = lax.iota(jnp.int32, 16)

    pltpu.sync_copy(eh_hbm, eh_v)
    pltpu.sync_copy(et_hbm, et_v)

    for c in range(BPB):
        def _zero_rows(i, _, c=c):
            for k in range(D // 16):
                rows[c][i, pl.ds(k * 16, 16)] = zero16
            return 0
        lax.fori_loop(0, K, _zero_rows, 0)

    def _zero_esl(i, _):
        for k in range(D // 16):
            esl_v[i, pl.ds(k * 16, 16)] = zero16
        return 0

    lax.fori_loop(0, EROWS, _zero_esl, 0)
    for k in range(EROWS // 16):
        idx_v[0, pl.ds(k * 16, 16)] = iota16 + (16 * k)

    for b in range(NPAD // NS // K):
        pltpu.sync_copy(r0, ftu_sh.at[pl.ds(sid * (NPAD // NS) + b * K,
                                            K), :])

    @pl.when(sid < EROWS // 8)
    def _():
        pltpu.sync_copy(r0.at[pl.ds(0, 8), :], esum_sh.at[pl.ds(sid * 8, 8), :])

    plsc.subcore_barrier()

    def _stage(t, q):
        sl = pl.ds(base_ch + t * SUP, SUP)
        a = pltpu.async_copy(ei_hbm.at[0, sl], src_v.at[q], ssem)
        b = pltpu.async_copy(ei_hbm.at[1, sl], dst_v.at[q], ssem)
        c = pltpu.async_copy(er_hbm.at[sl], er_v.at[q], ssem)
        del a, b, c

    def _wait_stage(q):
        pltpu.make_async_copy(ei_hbm.at[0, pl.ds(0, SUP)],
                              src_v.at[q], ssem).wait()
        pltpu.make_async_copy(ei_hbm.at[1, pl.ds(0, SUP)],
                              dst_v.at[q], ssem).wait()
        pltpu.make_async_copy(er_hbm.at[pl.ds(0, SUP)],
                              er_v.at[q], ssem).wait()

    _stage(0, 0)
    _wait_stage(0)
    for c in range(BPB):
        pltpu.async_copy(h_hbm.at[src_v.at[0, c]], rows[c], gsem[c])

    def _body(s, carry):
        t = s // 4
        q = t & 1
        m = s % 4
        qn = ((s + 1) // 4) & 1
        basen = BPB * ((s + 1) % 4)

        @pl.when((m == 0) & (s < nb - 4))
        def _():
            _stage(t + 1, 1 - q)

        @pl.when((m == 3) & (s < nb - 1))
        def _():
            _wait_stage(1 - q)

        for c in range(BPB):
            jj = BPB * m + c
            for g in range(K // 16):
                sl = pl.ds(g * 16, 16)
                sv = src_v[q, jj, sl]
                dv = dst_v[q, jj, sl]
                dvg = jnp.minimum(dv, N - 1)
                x = (plsc.load_gather(eh_v, [sv])
                     + plsc.load_gather(et_v, [dvg])
                     + er_v[q, jj, sl])
                x = jnp.where(x >= 0.0, x, 0.2 * x)
                w = jnp.exp(x)
                w_v[0, sl] = w
                plsc.addupdate_scatter(
                    esl_v,
                    [lax.shift_right_logical(dv, 7),
                     jnp.bitwise_and(dv, 127)],
                    w)
            pltpu.make_async_copy(h_hbm.at[src_v.at[q, jj]], rows[c],
                                  gsem[c]).wait()
            for g in range(K // 16):
                wvec = w_v[0, pl.ds(g * 16, 16)]
                for tt in range(16):
                    i = g * 16 + tt
                    wv = jnp.full((16,), wvec[tt], jnp.float32)
                    for k in range(D // 16):
                        sl = pl.ds(k * 16, 16)
                        rows[c][i, sl] = rows[c][i, sl] * wv
            pltpu.sync_copy(rows[c], ftu_sh.at[dst_v.at[q, jj]], add=True)

            @pl.when(s < nb - 1)
            def _(c=c, qn=qn, basen=basen):
                pltpu.async_copy(h_hbm.at[src_v.at[qn, basen + c]], rows[c],
                                 gsem[c])

        return carry

    lax.fori_loop(0, nb, _body, 0)

    pltpu.sync_copy(esl_v, esum_sh.at[idx_v.at[0]], add=True)
    plsc.subcore_barrier()

    for b in range(NPAD // NS // K):
        sl = pl.ds(sid * (NPAD // NS) + b * K, K)
        pltpu.sync_copy(ftu_sh.at[sl, :], r0)
        pltpu.sync_copy(r0, ftu_out.at[cid, sl, :])

    @pl.when(sid < EROWS // 8)
    def _():
        sl = pl.ds(sid * 8, 8)
        pltpu.sync_copy(esum_sh.at[sl, :], esl_v.at[pl.ds(0, 8), :])
        pltpu.sync_copy(esl_v.at[pl.ds(0, 8), :], esum_out.at[cid, sl, :])


def _make_sc_call():
    mesh = plsc.VectorSubcoreMesh(
        core_axis_name="c", subcore_axis_name="s",
        num_cores=NC, num_subcores=NS)

    return pl.kernel(
        _sc_body,
        out_type=[jax.ShapeDtypeStruct((NC, NPAD, D), jnp.float32),
                  jax.ShapeDtypeStruct((NC, EROWS, D), jnp.float32)],
        mesh=mesh,
        compiler_params=pltpu.CompilerParams(needs_layout_passes=False),
        scratch_types=[
            pltpu.VMEM((N,), jnp.float32),
            pltpu.VMEM((N,), jnp.float32),
            pltpu.VMEM((2, SUP, K), jnp.int32),
            pltpu.VMEM((2, SUP, K), jnp.int32),
            pltpu.VMEM((2, SUP, K), jnp.float32),
            pltpu.VMEM((1, K), jnp.float32),
            pltpu.VMEM((K, D), jnp.float32),
            pltpu.VMEM((K, D), jnp.float32),
            pltpu.VMEM((EROWS, D), jnp.float32),
            pltpu.VMEM((1, EROWS), jnp.int32),
            pltpu.VMEM_SHARED((NPAD, D), jnp.float32),
            pltpu.VMEM_SHARED((EROWS, D), jnp.float32),
            pltpu.SemaphoreType.DMA,
            pltpu.SemaphoreType.DMA,
            pltpu.SemaphoreType.DMA,
        ],
    )


def _final_body(ftu_ref, esum_ref, fcw_ref, fcb_ref, out_ref):
    B = 1024
    ftu = ftu_ref[0] + ftu_ref[1]
    es2 = esum_ref[0] + esum_ref[1]
    rr = lax.broadcasted_iota(jnp.int32, (B, B // D), 0) // D
    gsel = (rr == lax.broadcasted_iota(jnp.int32, (B, B // D), 1))
    grows = lax.dot_general(gsel.astype(jnp.float32), es2,
                            (((1,), (0,)), ((), ())),
                            preferred_element_type=jnp.float32)
    cc = lax.broadcasted_iota(jnp.int32, (B, D), 0) % D
    csel = (cc == lax.broadcasted_iota(jnp.int32, (B, D), 1))
    ones = jnp.ones((1, D), jnp.float32)
    es = lax.dot_general(jnp.where(csel, grows, 0.0), ones,
                         (((1,), (1,)), ((), ())),
                         preferred_element_type=jnp.float32)
    o = lax.dot_general(ftu, fcw_ref[...], (((1,), (1,)), ((), ())),
                        preferred_element_type=jnp.float32)
    o = o / jnp.maximum(es, 1e-20) + fcb_ref[...]
    norm = jnp.sqrt(jnp.sum(o * o, axis=1, keepdims=True))
    out_ref[...] = o / jnp.maximum(norm, 1e-12)


def kernel(h, r, edge_index, head_W, tail_W, rel_W, fc_W, fc_b,
           ha2, hb2, ta2, tb2, ra2, rb2):
    f32 = jnp.float32
    eh2, et2 = pl.pallas_call(
        _node_logits_body,
        grid=(N // 2000,),
        in_specs=[
            pl.BlockSpec((2000, D), lambda i: (i, 0)),
            pl.BlockSpec((1, D), lambda i: (0, 0)),
            pl.BlockSpec((1, D), lambda i: (0, 0)),
            pl.BlockSpec((1, D), lambda i: (0, 0)),
            pl.BlockSpec((1, D), lambda i: (0, 0)),
            pl.BlockSpec((1, D), lambda i: (0, 0)),
            pl.BlockSpec((1, D), lambda i: (0, 0)),
        ],
        out_specs=[pl.BlockSpec((2000, 1), lambda i: (i, 0))] * 2,
        out_shape=[jax.ShapeDtypeStruct((N, 1), f32)] * 2,
    )(h, ha2.reshape(1, D), hb2.reshape(1, D), head_W,
      ta2.reshape(1, D), tb2.reshape(1, D), tail_W)

    er2 = pl.pallas_call(
        _edge_logits_body,
        grid=(E // 4000,),
        in_specs=[
            pl.BlockSpec((4000, D), lambda i: (i, 0)),
            pl.BlockSpec((1, D), lambda i: (0, 0)),
            pl.BlockSpec((1, D), lambda i: (0, 0)),
            pl.BlockSpec((1, D), lambda i: (0, 0)),
        ],
        out_specs=pl.BlockSpec((4000, 1), lambda i: (i, 0)),
        out_shape=jax.ShapeDtypeStruct((E, 1), f32),
    )(r, ra2.reshape(1, D), rb2.reshape(1, D), rel_W)

    pad = EPAD - E
    ei_pad = jnp.concatenate(
        [jnp.zeros((1, pad), jnp.int32), jnp.full((1, pad), N, jnp.int32)])
    ei = jnp.concatenate([edge_index, ei_pad], axis=1).reshape(2, NCH, K)
    er = jnp.concatenate([er2, jnp.zeros((pad, 1), f32)]).reshape(NCH, K)

    ftu_p, esum_p = _make_sc_call()(ei, er, eh2.reshape(N), et2.reshape(N), h)

    out = pl.pallas_call(
        _final_body,
        grid=(NPAD // 1024,),
        in_specs=[
            pl.BlockSpec((NC, 1024, D), lambda i: (0, i, 0)),
            pl.BlockSpec((NC, 8, D), lambda i: (0, i, 0)),
            pl.BlockSpec((D, D), lambda i: (0, 0)),
            pl.BlockSpec((1, D), lambda i: (0, 0)),
        ],
        out_specs=pl.BlockSpec((1024, D), lambda i: (i, 0)),
        out_shape=jax.ShapeDtypeStruct((N, D), f32),
    )(ftu_p, esum_p, fc_W, fc_b.reshape(1, D))
    return out
